# baseline (speedup 1.0000x reference)
import functools

import jax
import jax.numpy as jnp
from jax.experimental import pallas as pl

_KT = 128


def _vq_body(x_ref, e_ref, res_ref, idx_ref, *, M, K, C):
    X = x_ref[0]
    Xs = X * -2.0
    x2 = jnp.sum(X * X, axis=1, keepdims=True)
    e2 = jnp.sum(e_ref[...] * e_ref[...], axis=1)[None, :]

    tiles = []
    for kt in range(K // _KT):
        E_t = e_ref[kt * _KT:(kt + 1) * _KT, :]
        p_t = jax.lax.dot_general(Xs, E_t, (((1,), (1,)), ((), ())),
                                  preferred_element_type=jnp.float32)
        tiles.append(jnp.min(p_t + x2, axis=0, keepdims=True))
    colming = jnp.concatenate(tiles, axis=1)
    colmin = colming + e2

    gmin = jnp.min(colmin)
    kiota = jax.lax.broadcasted_iota(jnp.int32, (1, K), 1)
    k_star = jnp.min(jnp.where(colmin == gmin, kiota, K))

    e_row = e_ref[pl.ds(k_star, 1), :]
    gcol = jnp.sum(Xs * e_row, axis=1, keepdims=True) + x2
    cmin = jnp.min(gcol)
    miota = jax.lax.broadcasted_iota(jnp.int32, (M, 1), 0)
    m_star = jnp.min(jnp.where(gcol == cmin, miota, M))

    res_ref[0] = x_ref[0, pl.ds(m_star, 1), :] - e_ref[pl.ds(k_star, 1), :]
    idx_ref[0] = jnp.reshape(k_star, (1, 1))


def kernel(x, argmin, last, embed):
    del argmin
    N, M, C = x.shape
    K = embed.shape[0]
    body = functools.partial(_vq_body, M=M, K=K, C=C)
    res, idx = pl.pallas_call(
        body,
        grid=(N,),
        in_specs=[
            pl.BlockSpec((1, M, C), lambda n: (n, 0, 0)),
            pl.BlockSpec((K, C), lambda n: (0, 0)),
        ],
        out_specs=[
            pl.BlockSpec((1, 1, C), lambda n: (n, 0, 0)),
            pl.BlockSpec((1, 1, 1), lambda n: (n, 0, 0)),
        ],
        out_shape=[
            jax.ShapeDtypeStruct((N, 1, C), x.dtype),
            jax.ShapeDtypeStruct((N, 1, 1), jnp.int32),
        ],
    )(x, embed)
    return res * jnp.asarray(last, x.dtype), idx.reshape(N, 1)


# speedup vs baseline: 1.1969x; 1.1969x over previous
import functools

import jax
import jax.numpy as jnp
from jax.experimental import pallas as pl

_KT = 256


def _vq_body(x_ref, es_ref, e2_ref, res_ref, idx_ref, *, M, K, C):
    X = x_ref[0]
    x2 = jnp.sum(X * X, axis=1, keepdims=True)

    tiles = []
    for kt in range(K // _KT):
        Es_t = es_ref[kt * _KT:(kt + 1) * _KT, :]
        p_t = jax.lax.dot_general(X, Es_t, (((1,), (1,)), ((), ())),
                                  preferred_element_type=jnp.float32)
        tiles.append(jnp.min(p_t + x2, axis=0, keepdims=True))
    colmin = jnp.concatenate(tiles, axis=1) + e2_ref[...]

    gmin = jnp.min(colmin)
    kiota = jax.lax.broadcasted_iota(jnp.int32, (1, K), 1)
    k_star = jnp.min(jnp.where(colmin == gmin, kiota, K))

    es_row = es_ref[pl.ds(k_star, 1), :]
    gcol = jnp.sum(X * es_row, axis=1, keepdims=True) + x2
    cmin = jnp.min(gcol)
    miota = jax.lax.broadcasted_iota(jnp.int32, (M, 1), 0)
    m_star = jnp.min(jnp.where(gcol == cmin, miota, M))

    res_ref[0] = x_ref[0, pl.ds(m_star, 1), :] + 0.5 * es_row
    idx_ref[0] = jnp.reshape(k_star, (1, 1))


def kernel(x, argmin, last, embed):
    del argmin
    N, M, C = x.shape
    K = embed.shape[0]
    es = embed * -2.0
    e2 = jnp.sum(embed * embed, axis=1)[None, :]
    body = functools.partial(_vq_body, M=M, K=K, C=C)
    res, idx = pl.pallas_call(
        body,
        grid=(N,),
        in_specs=[
            pl.BlockSpec((1, M, C), lambda n: (n, 0, 0)),
            pl.BlockSpec((K, C), lambda n: (0, 0)),
            pl.BlockSpec((1, K), lambda n: (0, 0)),
        ],
        out_specs=[
            pl.BlockSpec((1, 1, C), lambda n: (n, 0, 0)),
            pl.BlockSpec((1, 1, 1), lambda n: (n, 0, 0)),
        ],
        out_shape=[
            jax.ShapeDtypeStruct((N, 1, C), x.dtype),
            jax.ShapeDtypeStruct((N, 1, 1), jnp.int32),
        ],
    )(x, es, e2)
    return res * jnp.asarray(last, x.dtype), idx.reshape(N, 1)


# speedup vs baseline: 1.6802x; 1.4038x over previous
import functools

import jax
import jax.numpy as jnp
from jax.experimental import pallas as pl
from jax.experimental.pallas import tpu as pltpu

_KT = 256
_NB = 2


def _vq_body(last_ref, x_ref, e_ref, res_ref, idx_ref, e2_ref, *, M, K, C):
    E = e_ref[...]

    @pl.when(pl.program_id(0) == 0)
    def _init():
        e2_ref[...] = jnp.sum(E * E, axis=1)[None, :]

    lastv = last_ref[...]

    for i in range(_NB):
        X = x_ref[i]
        hx2 = 0.5 * jnp.sum(X * X, axis=1, keepdims=True)

        tiles = []
        for kt in range(K // _KT):
            E_t = e_ref[kt * _KT:(kt + 1) * _KT, :]
            p_t = jax.lax.dot_general(X, E_t, (((1,), (1,)), ((), ())),
                                      preferred_element_type=jnp.float32)
            tiles.append(jnp.min(hx2 - p_t, axis=0, keepdims=True))
        colmin = 2.0 * jnp.concatenate(tiles, axis=1) + e2_ref[...]

        gmin = jnp.min(colmin)
        kiota = jax.lax.broadcasted_iota(jnp.int32, (1, K), 1)
        k_star = jnp.min(jnp.where(colmin == gmin, kiota, K))

        e_row = e_ref[pl.ds(k_star, 1), :]
        gcol = hx2 - jnp.sum(X * e_row, axis=1, keepdims=True)
        cmin = jnp.min(gcol)
        miota = jax.lax.broadcasted_iota(jnp.int32, (M, 1), 0)
        m_star = jnp.min(jnp.where(gcol == cmin, miota, M))

        res_ref[i] = (x_ref[i, pl.ds(m_star, 1), :] - e_row) * lastv
        idx_ref[i] = jnp.reshape(k_star, (1, 1))


def kernel(x, argmin, last, embed):
    del argmin
    N, M, C = x.shape
    K = embed.shape[0]
    lastv = jnp.asarray(last, x.dtype).reshape(1, 1)
    body = functools.partial(_vq_body, M=M, K=K, C=C)
    res, idx = pl.pallas_call(
        body,
        grid=(N // _NB,),
        in_specs=[
            pl.BlockSpec((1, 1), lambda n: (0, 0)),
            pl.BlockSpec((_NB, M, C), lambda n: (n, 0, 0)),
            pl.BlockSpec((K, C), lambda n: (0, 0)),
        ],
        out_specs=[
            pl.BlockSpec((_NB, 1, C), lambda n: (n, 0, 0)),
            pl.BlockSpec((_NB, 1, 1), lambda n: (n, 0, 0)),
        ],
        out_shape=[
            jax.ShapeDtypeStruct((N, 1, C), x.dtype),
            jax.ShapeDtypeStruct((N, 1, 1), jnp.int32),
        ],
        scratch_shapes=[pltpu.VMEM((1, K), jnp.float32)],
    )(lastv, x, embed)
    return res, idx.reshape(N, 1)


# speedup vs baseline: 1.9111x; 1.1374x over previous
import functools

import jax
import jax.numpy as jnp
from jax.experimental import pallas as pl
from jax.experimental.pallas import tpu as pltpu

_KT = 256
_NB = 4


def _vq_body(last_ref, x_ref, e_ref, res_ref, idx_ref, e2_ref, *, M, K, C):
    E = e_ref[...]

    @pl.when(pl.program_id(0) == 0)
    def _init():
        e2_ref[...] = jnp.sum(E * E, axis=1)[None, :]

    lastv = last_ref[...]

    for i in range(_NB):
        X = x_ref[i]
        hx2 = 0.5 * jnp.sum(X * X, axis=1, keepdims=True)

        tiles = []
        for kt in range(K // _KT):
            E_t = e_ref[kt * _KT:(kt + 1) * _KT, :]
            p_t = jax.lax.dot_general(X, E_t, (((1,), (1,)), ((), ())),
                                      preferred_element_type=jnp.float32)
            tiles.append(jnp.min(hx2 - p_t, axis=0, keepdims=True))
        colmin = 2.0 * jnp.concatenate(tiles, axis=1) + e2_ref[...]

        gmin = jnp.min(colmin)
        kiota = jax.lax.broadcasted_iota(jnp.int32, (1, K), 1)
        k_star = jnp.min(jnp.where(colmin == gmin, kiota, K))

        e_row = e_ref[pl.ds(k_star, 1), :]
        gcol = hx2 - jnp.sum(X * e_row, axis=1, keepdims=True)
        cmin = jnp.min(gcol)
        miota = jax.lax.broadcasted_iota(jnp.int32, (M, 1), 0)
        m_star = jnp.min(jnp.where(gcol == cmin, miota, M))

        res_ref[i] = (x_ref[i, pl.ds(m_star, 1), :] - e_row) * lastv
        idx_ref[i] = jnp.reshape(k_star, (1, 1))


def kernel(x, argmin, last, embed):
    del argmin
    N, M, C = x.shape
    K = embed.shape[0]
    lastv = jnp.asarray(last, x.dtype).reshape(1, 1)
    body = functools.partial(_vq_body, M=M, K=K, C=C)
    res, idx = pl.pallas_call(
        body,
        grid=(N // _NB,),
        in_specs=[
            pl.BlockSpec((1, 1), lambda n: (0, 0)),
            pl.BlockSpec((_NB, M, C), lambda n: (n, 0, 0)),
            pl.BlockSpec((K, C), lambda n: (0, 0)),
        ],
        out_specs=[
            pl.BlockSpec((_NB, 1, C), lambda n: (n, 0, 0)),
            pl.BlockSpec((_NB, 1, 1), lambda n: (n, 0, 0)),
        ],
        out_shape=[
            jax.ShapeDtypeStruct((N, 1, C), x.dtype),
            jax.ShapeDtypeStruct((N, 1, 1), jnp.int32),
        ],
        scratch_shapes=[pltpu.VMEM((1, K), jnp.float32)],
    )(lastv, x, embed)
    return res, idx.reshape(N, 1)


# speedup vs baseline: 1.9671x; 1.0293x over previous
import functools

import jax
import jax.numpy as jnp
from jax.experimental import pallas as pl
from jax.experimental.pallas import tpu as pltpu

_KT = 256
_NB = 8


def _vq_body(last_ref, x_ref, e_ref, res_ref, idx_ref, e2_ref, *, M, K, C):
    E = e_ref[...]

    @pl.when(pl.program_id(0) == 0)
    def _init():
        e2_ref[...] = jnp.sum(E * E, axis=1)[None, :]

    lastv = last_ref[...]

    for i in range(_NB):
        X = x_ref[i]
        hx2 = 0.5 * jnp.sum(X * X, axis=1, keepdims=True)

        tiles = []
        for kt in range(K // _KT):
            E_t = e_ref[kt * _KT:(kt + 1) * _KT, :]
            p_t = jax.lax.dot_general(X, E_t, (((1,), (1,)), ((), ())),
                                      preferred_element_type=jnp.float32)
            tiles.append(jnp.min(hx2 - p_t, axis=0, keepdims=True))
        colmin = 2.0 * jnp.concatenate(tiles, axis=1) + e2_ref[...]

        gmin = jnp.min(colmin)
        kiota = jax.lax.broadcasted_iota(jnp.int32, (1, K), 1)
        k_star = jnp.min(jnp.where(colmin == gmin, kiota, K))

        e_row = e_ref[pl.ds(k_star, 1), :]
        gcol = hx2 - jnp.sum(X * e_row, axis=1, keepdims=True)
        cmin = jnp.min(gcol)
        miota = jax.lax.broadcasted_iota(jnp.int32, (M, 1), 0)
        m_star = jnp.min(jnp.where(gcol == cmin, miota, M))

        res_ref[i] = (x_ref[i, pl.ds(m_star, 1), :] - e_row) * lastv
        idx_ref[i] = jnp.reshape(k_star, (1, 1))


def kernel(x, argmin, last, embed):
    del argmin
    N, M, C = x.shape
    K = embed.shape[0]
    lastv = jnp.asarray(last, x.dtype).reshape(1, 1)
    body = functools.partial(_vq_body, M=M, K=K, C=C)
    res, idx = pl.pallas_call(
        body,
        grid=(N // _NB,),
        in_specs=[
            pl.BlockSpec((1, 1), lambda n: (0, 0)),
            pl.BlockSpec((_NB, M, C), lambda n: (n, 0, 0)),
            pl.BlockSpec((K, C), lambda n: (0, 0)),
        ],
        out_specs=[
            pl.BlockSpec((_NB, 1, C), lambda n: (n, 0, 0)),
            pl.BlockSpec((_NB, 1, 1), lambda n: (n, 0, 0)),
        ],
        out_shape=[
            jax.ShapeDtypeStruct((N, 1, C), x.dtype),
            jax.ShapeDtypeStruct((N, 1, 1), jnp.int32),
        ],
        scratch_shapes=[pltpu.VMEM((1, K), jnp.float32)],
    )(lastv, x, embed)
    return res, idx.reshape(N, 1)


# speedup vs baseline: 1.9758x; 1.0044x over previous
import functools

import jax
import jax.numpy as jnp
from jax.experimental import pallas as pl
from jax.experimental.pallas import tpu as pltpu

_KT = 256
_NB = 8


def _vq_body(last_ref, x_ref, e_ref, res_ref, idx_ref, *, M, K, C):
    E = e_ref[...]
    e2 = jnp.sum(E * E, axis=1)[None, :]
    lastv = last_ref[...]

    for i in range(_NB):
        X = x_ref[i]
        hx2 = 0.5 * jnp.sum(X * X, axis=1, keepdims=True)

        tiles = []
        for kt in range(K // _KT):
            E_t = e_ref[kt * _KT:(kt + 1) * _KT, :]
            p_t = jax.lax.dot_general(X, E_t, (((1,), (1,)), ((), ())),
                                      preferred_element_type=jnp.float32)
            tiles.append(jnp.min(hx2 - p_t, axis=0, keepdims=True))
        colmin = 2.0 * jnp.concatenate(tiles, axis=1) + e2

        gmin = jnp.min(colmin)
        kiota = jax.lax.broadcasted_iota(jnp.int32, (1, K), 1)
        k_star = jnp.min(jnp.where(colmin == gmin, kiota, K))

        e_row = e_ref[pl.ds(k_star, 1), :]
        gcol = hx2 - jnp.sum(X * e_row, axis=1, keepdims=True)
        cmin = jnp.min(gcol)
        miota = jax.lax.broadcasted_iota(jnp.int32, (M, 1), 0)
        m_star = jnp.min(jnp.where(gcol == cmin, miota, M))

        res_ref[i] = (x_ref[i, pl.ds(m_star, 1), :] - e_row) * lastv
        idx_ref[i] = jnp.reshape(k_star, (1, 1))


def kernel(x, argmin, last, embed):
    del argmin
    N, M, C = x.shape
    K = embed.shape[0]
    lastv = jnp.asarray(last, x.dtype).reshape(1, 1)
    body = functools.partial(_vq_body, M=M, K=K, C=C)
    res, idx = pl.pallas_call(
        body,
        grid=(N // _NB,),
        in_specs=[
            pl.BlockSpec((1, 1), lambda n: (0, 0)),
            pl.BlockSpec((_NB, M, C), lambda n: (n, 0, 0)),
            pl.BlockSpec((K, C), lambda n: (0, 0)),
        ],
        out_specs=[
            pl.BlockSpec((_NB, 1, C), lambda n: (n, 0, 0)),
            pl.BlockSpec((_NB, 1, 1), lambda n: (n, 0, 0)),
        ],
        out_shape=[
            jax.ShapeDtypeStruct((N, 1, C), x.dtype),
            jax.ShapeDtypeStruct((N, 1, 1), jnp.int32),
        ],
        compiler_params=pltpu.CompilerParams(
            dimension_semantics=("parallel",)),
    )(lastv, x, embed)
    return res, idx.reshape(N, 1)


# speedup vs baseline: 2.0158x; 1.0202x over previous
import functools

import jax
import jax.numpy as jnp
from jax.experimental import pallas as pl
from jax.experimental.pallas import tpu as pltpu

_KT = 256
_NB = 8


def _vq_body(last_ref, x_ref, e_ref, res_ref, idx_ref, *, M, K, C):
    E = e_ref[...]
    e2 = jnp.sum(E * E, axis=1)[None, :]
    lastv = last_ref[...]

    for i in range(_NB):
        X = x_ref[i]
        hx2 = 0.5 * jnp.sum(X * X, axis=1, keepdims=True)

        tiles = []
        for kt in range(K // _KT):
            E_t = e_ref[kt * _KT:(kt + 1) * _KT, :]
            p_t = jax.lax.dot_general(X, E_t, (((1,), (1,)), ((), ())),
                                      preferred_element_type=jnp.float32)
            tiles.append(jnp.min(hx2 - p_t, axis=0, keepdims=True))
        colmin = 2.0 * jnp.concatenate(tiles, axis=1) + e2

        gmin = jnp.min(colmin)
        kiota = jax.lax.broadcasted_iota(jnp.int32, (1, K), 1)
        k_star = jnp.min(jnp.where(colmin == gmin, kiota, K))

        e_row = e_ref[pl.ds(k_star, 1), :]
        gcol = hx2 - jnp.sum(X * e_row, axis=1, keepdims=True)
        cmin = jnp.min(gcol)
        miota = jax.lax.broadcasted_iota(jnp.int32, (M, 1), 0)
        m_star = jnp.min(jnp.where(gcol == cmin, miota, M))

        res_ref[i] = (x_ref[i, pl.ds(m_star, 1), :] - e_row) * lastv
        idx_ref[pl.ds(i, 1), :] = jnp.reshape(k_star, (1, 1))


def kernel(x, argmin, last, embed):
    del argmin
    N, M, C = x.shape
    K = embed.shape[0]
    lastv = jnp.asarray(last, x.dtype).reshape(1, 1)
    body = functools.partial(_vq_body, M=M, K=K, C=C)
    res, idx = pl.pallas_call(
        body,
        grid=(N // _NB,),
        in_specs=[
            pl.BlockSpec((1, 1), lambda n: (0, 0)),
            pl.BlockSpec((_NB, M, C), lambda n: (n, 0, 0)),
            pl.BlockSpec((K, C), lambda n: (0, 0)),
        ],
        out_specs=[
            pl.BlockSpec((_NB, 1, C), lambda n: (n, 0, 0)),
            pl.BlockSpec((_NB, 1), lambda n: (n, 0)),
        ],
        out_shape=[
            jax.ShapeDtypeStruct((N, 1, C), x.dtype),
            jax.ShapeDtypeStruct((N, 1), jnp.int32),
        ],
        compiler_params=pltpu.CompilerParams(
            dimension_semantics=("parallel",)),
    )(lastv, x, embed)
    return res, idx
